# baseline (speedup 1.0000x reference)
import functools

import jax
import jax.numpy as jnp
from jax import lax
from jax.experimental import pallas as pl
from jax.experimental.pallas import tpu as pltpu
from jax.experimental.pallas import tpu_sc as plsc

NN = 10000
NNP = 10240
EE = 320000
HH = 128
GG = 64
DP = 16
CW = 128
NROWS = 2560
EEP = NROWS * CW
NC = 2
NS = 16
NW = NC * NS
RW = NROWS // NW
RWC = (24, 136)
RWMAX = max(RWC)
CBASE = (0, NS * RWC[0])
ZR = 128
NPW = NNP // NS
MF = GG * NNP
MPW = MF // NS
MZC = 4096


def _sc_mesh():
  return plsc.VectorSubcoreMesh(core_axis_name="c", subcore_axis_name="s",
                                num_cores=NC, num_subcores=NS)


@functools.cache
def _deg_kernel():
  @functools.partial(
      pl.kernel,
      out_type=jax.ShapeDtypeStruct((NC, NNP), jnp.float32),
      mesh=_sc_mesh(),
      compiler_params=pltpu.CompilerParams(use_tc_tiling_on_sc=False),
      scratch_types=[
          pltpu.VMEM((RW * CW,), jnp.int32),
          pltpu.VMEM((RW * CW,), jnp.float32),
          pltpu.VMEM((NPW,), jnp.float32),
          pltpu.VMEM_SHARED((NNP,), jnp.float32),
      ],
  )
  def deg(dst_hbm, out_hbm, dstv, onesv, zb, acc):
    c = lax.axis_index("c")
    s = lax.axis_index("s")
    wid = c * NS + s

    def fill1(i, _):
      onesv[pl.ds(i * 16, 16)] = jnp.ones((16,), jnp.float32)
      return 0
    lax.fori_loop(0, RW * CW // 16, fill1, 0)

    def fill0(i, _):
      zb[pl.ds(i * 16, 16)] = jnp.zeros((16,), jnp.float32)
      return 0
    lax.fori_loop(0, NPW // 16, fill0, 0)

    pltpu.sync_copy(dst_hbm.at[pl.ds(wid * RW * CW, RW * CW)], dstv)
    pltpu.sync_copy(zb, acc.at[pl.ds(s * NPW, NPW)])
    plsc.subcore_barrier()

    pltpu.sync_copy(onesv, acc.at[dstv], add=True)

    plsc.subcore_barrier()
    pltpu.sync_copy(acc.at[pl.ds(s * NPW, NPW)],
                    out_hbm.at[c, pl.ds(s * NPW, NPW)])

  return deg


@functools.cache
def _prop_kernel(F):
  @functools.partial(
      pl.kernel,
      out_type=jax.ShapeDtypeStruct((NC, NNP, F), jnp.float32),
      mesh=_sc_mesh(),
      compiler_params=pltpu.CompilerParams(use_tc_tiling_on_sc=False),
      scratch_types=[
          pltpu.VMEM((RWMAX, CW), jnp.int32),
          pltpu.VMEM((RWMAX, CW), jnp.int32),
          pltpu.VMEM((CW, F), jnp.float32),
          pltpu.VMEM((CW, F), jnp.float32),
          pltpu.VMEM((CW, F), jnp.float32),
          pltpu.VMEM((CW, F), jnp.float32),
          pltpu.VMEM((ZR, F), jnp.float32),
          pltpu.VMEM_SHARED((NNP, F), jnp.float32),
          pltpu.SemaphoreType.DMA,
          pltpu.SemaphoreType.DMA,
          pltpu.SemaphoreType.DMA,
          pltpu.SemaphoreType.DMA,
          pltpu.SemaphoreType.DMA,
          pltpu.SemaphoreType.DMA,
          pltpu.SemaphoreType.DMA,
          pltpu.SemaphoreType.DMA,
      ],
  )
  def prop(h_hbm, src_hbm, dst_hbm, out_hbm, srcv, dstv, rb0, rb1, rb2, rb3,
           zb, acc, sg0, sg1, sg2, sg3, ss0, ss1, ss2, ss3):
    c = lax.axis_index("c")
    s = lax.axis_index("s")
    rbufs = (rb0, rb1, rb2, rb3)
    sgs = (sg0, sg1, sg2, sg3)
    sss = (ss0, ss1, ss2, ss3)

    def z0(i, _):
      def z1(k, _):
        zb[i, pl.ds(k * 16, 16)] = jnp.zeros((16,), jnp.float32)
        return 0
      lax.fori_loop(0, F // 16, z1, 0)
      return 0
    lax.fori_loop(0, ZR, z0, 0)

    def zc(t, _):
      pltpu.sync_copy(zb, acc.at[pl.ds(s * NPW + t * ZR, ZR)])
      return 0
    lax.fori_loop(0, NPW // ZR, zc, 0)

    base = jnp.where(c == 0, s * RWC[0], CBASE[1] + s * RWC[1])
    pltpu.sync_copy(src_hbm.at[pl.ds(base, RWMAX)], srcv)
    pltpu.sync_copy(dst_hbm.at[pl.ds(base, RWMAX)], dstv)
    plsc.subcore_barrier()

    def run(cc):
      rows = RWC[cc]

      pltpu.async_copy(h_hbm.at[srcv.at[0]], rb0, sg0)
      pltpu.async_copy(h_hbm.at[srcv.at[1]], rb1, sg1)

      def body(j, _):
        def make_branch(b):
          b2 = (b + 2) % 4

          def br():
            pltpu.make_async_copy(h_hbm.at[srcv.at[j]], rbufs[b],
                                  sgs[b]).wait()
            pltpu.async_copy(rbufs[b], acc.at[dstv.at[j]], sss[b], add=True)

            @pl.when(j + 2 < rows)
            def _():
              @pl.when(j >= 2)
              def _():
                pltpu.make_async_copy(rbufs[b2], acc.at[dstv.at[j - 2]],
                                      sss[b2]).wait()
              pltpu.async_copy(h_hbm.at[srcv.at[j + 2]], rbufs[b2], sgs[b2])
          return br

        for b in range(4):
          pl.when(j % 4 == b)(make_branch(b))
        return 0
      lax.fori_loop(0, rows, body, 0)

      for b in range(4):
        pltpu.make_async_copy(rbufs[(rows - 4 + b) % 4],
                              acc.at[dstv.at[rows - 4 + b]],
                              sss[(rows - 4 + b) % 4]).wait()

    for cc in range(NC):
      pl.when(c == cc)(lambda cc=cc: run(cc))

    plsc.subcore_barrier()

    def wo(t, _):
      off = s * NPW + t * ZR
      pltpu.sync_copy(acc.at[pl.ds(off, ZR)], out_hbm.at[c, pl.ds(off, ZR)])
      return 0
    lax.fori_loop(0, NPW // ZR, wo, 0)

  return prop


@functools.cache
def _mbuild_kernel():
  @functools.partial(
      pl.kernel,
      out_type=jax.ShapeDtypeStruct((NC, MF), jnp.float32),
      mesh=_sc_mesh(),
      compiler_params=pltpu.CompilerParams(use_tc_tiling_on_sc=False,
                                           needs_layout_passes=False),
      scratch_types=[
          pltpu.VMEM((NNP,), jnp.float32),
          pltpu.VMEM((NNP,), jnp.float32),
          pltpu.VMEM((NNP,), jnp.int32),
          pltpu.VMEM((RW * CW,), jnp.int32),
          pltpu.VMEM((RW * CW,), jnp.int32),
          pltpu.VMEM((RW * CW,), jnp.float32),
          pltpu.VMEM((RW * CW,), jnp.int32),
          pltpu.VMEM((MZC,), jnp.float32),
          pltpu.VMEM_SHARED((MF,), jnp.float32),
      ],
  )
  def mbuild(src_hbm, dst_hbm, dinv_hbm, dinvc_hbm, fiv_hbm, out_hbm,
             dinvv, dinvcv, fivv, srcv, dstv, vals, fidx, zb, macc):
    c = lax.axis_index("c")
    s = lax.axis_index("s")

    wid = c * NS + s

    def fz(i, _):
      zb[pl.ds(i * 16, 16)] = jnp.zeros((16,), jnp.float32)
      return 0
    lax.fori_loop(0, MZC // 16, fz, 0)

    pltpu.sync_copy(dinv_hbm, dinvv)
    pltpu.sync_copy(dinvc_hbm, dinvcv)
    pltpu.sync_copy(fiv_hbm, fivv)
    pltpu.sync_copy(src_hbm.at[pl.ds(wid * RW * CW, RW * CW)], srcv)
    pltpu.sync_copy(dst_hbm.at[pl.ds(wid * RW * CW, RW * CW)], dstv)

    def zc(t, _):
      pltpu.sync_copy(zb, macc.at[pl.ds(s * MPW + t * MZC, MZC)])
      return 0
    lax.fori_loop(0, MPW // MZC, zc, 0)

    plsc.subcore_barrier()

    def body(i, _):
      sl = pl.ds(i * 16, 16)
      s16 = srcv[sl]
      d16 = dstv[sl]
      a = plsc.load_gather(dinvv, [s16])
      b = plsc.load_gather(dinvcv, [d16])
      f = plsc.load_gather(fivv, [d16])
      vals[sl] = a * b
      fidx[sl] = f + s16
      return 0
    lax.fori_loop(0, RW * CW // 16, body, 0)
    pltpu.sync_copy(vals, macc.at[fidx], add=True)

    plsc.subcore_barrier()

    def wo(t, _):
      off = s * MPW + t * MZC
      pltpu.sync_copy(macc.at[pl.ds(off, MZC)], out_hbm.at[c, pl.ds(off, MZC)])
      return 0
    lax.fori_loop(0, MPW // MZC, wo, 0)

  return mbuild


def _stats_tc(degp, batch_col, xp):
  def body(degp_ref, b_ref, x_ref, dinv_ref, xs_ref, dinvc_ref, fiv_ref,
           invc_ref, mask_ref):
    deg = degp_ref[0] + degp_ref[1] + 1.0
    dinv = lax.rsqrt(deg)
    b = b_ref[:]
    io = lax.broadcasted_iota(jnp.int32, (NN, GG), 1)
    oh = (b == io).astype(jnp.float32)
    cnt = jnp.sum(oh, axis=0, keepdims=True)
    invc = 1.0 / jnp.maximum(cnt, 1.0)
    ipn = lax.dot_general(oh, invc, (((1,), (1,)), ((), ())),
                          preferred_element_type=jnp.float32)
    dinv_ref[:] = dinv
    xs_ref[:] = x_ref[:] * dinv
    dinvc_ref[:] = dinv * ipn
    fiv_ref[:] = b * NNP
    invc_ref[:] = invc
    mask_ref[:] = (cnt > 0.0).astype(jnp.float32)

  return pl.pallas_call(
      body,
      out_shape=(
          jax.ShapeDtypeStruct((NN, 1), jnp.float32),
          jax.ShapeDtypeStruct((NN, DP), jnp.float32),
          jax.ShapeDtypeStruct((NN, 1), jnp.float32),
          jax.ShapeDtypeStruct((NN, 1), jnp.int32),
          jax.ShapeDtypeStruct((1, GG), jnp.float32),
          jax.ShapeDtypeStruct((1, GG), jnp.float32),
      ),
  )(degp, batch_col, xp)


def _layer_tc(p, hprev, dinv_col, Wm, bias, relu, scale_out):
  Fout = HH if Wm is None else Wm.shape[1]

  def body(p_ref, h_ref, d_ref, *rest):
    if Wm is None:
      b_ref, o_ref = rest
    else:
      w_ref, b_ref, o_ref = rest
    z = (p_ref[0] + p_ref[1] + h_ref[:]) * d_ref[:]
    if Wm is not None:
      z = jnp.dot(z, w_ref[:], preferred_element_type=jnp.float32)
    h = z + b_ref[:]
    if relu:
      h = jnp.maximum(h, 0.0)
    if scale_out:
      h = h * d_ref[:]
    o_ref[:] = h

  args = (p, hprev, dinv_col) + (() if Wm is None else (Wm,)) + (bias,)
  return pl.pallas_call(
      body, out_shape=jax.ShapeDtypeStruct((NN, Fout), jnp.float32)
  )(*args)


def _layer2_tc(p_lo, p_hi, hprev, dinv_col, Wm, bias):
  def body(plo_ref, phi_ref, h_ref, d_ref, w_ref, b_ref, o_ref):
    p = jnp.concatenate([plo_ref[0] + plo_ref[1], phi_ref[0] + phi_ref[1]],
                        axis=1)
    z = (p + h_ref[:]) * d_ref[:]
    h = jnp.dot(z, w_ref[:], preferred_element_type=jnp.float32) + b_ref[:]
    o_ref[:] = jnp.maximum(h, 0.0)

  return pl.pallas_call(
      body, out_shape=jax.ShapeDtypeStruct((NN, HH), jnp.float32)
  )(p_lo, p_hi, hprev, dinv_col, Wm, bias)


def _head_tc(Mp, h2, batch_row, dinv_row, invc_col, mask_col, W3, b3r, Wl, blr):
  def body(mp_ref, h2_ref, b_ref, dv_ref, ic_ref, mk_ref, w3_ref, b3_ref,
           wl_ref, bl_ref, o_ref):
    io = lax.broadcasted_iota(jnp.int32, (GG, NN), 0)
    oh = (b_ref[:] == io).astype(jnp.float32)
    dv = dv_ref[:]
    mtot = mp_ref[0] + mp_ref[1] + oh * (dv * dv) * ic_ref[:]
    pooled = jnp.dot(mtot, h2_ref[:], preferred_element_type=jnp.float32)
    pooled = (jnp.dot(pooled, w3_ref[:], preferred_element_type=jnp.float32)
              + mk_ref[:] * b3_ref[:])
    o_ref[:] = (jnp.dot(pooled, wl_ref[:], preferred_element_type=jnp.float32)
                + bl_ref[:])

  return pl.pallas_call(
      body, out_shape=jax.ShapeDtypeStruct((GG, 2), jnp.float32)
  )(Mp, h2, batch_row, dinv_row, invc_col, mask_col, W3, b3r, Wl, blr)


def _pad_rows(a, rows):
  return jnp.pad(a, ((0, rows - a.shape[0]),) + ((0, 0),) * (a.ndim - 1))


def kernel(x, edge_index, batch, W1, b1, W2, b2, W3, b3, Wl, bl):
  pad_e = jnp.full((EEP - EE,), NN, jnp.int32)
  src1 = jnp.concatenate([edge_index[0], pad_e])
  dst1 = jnp.concatenate([edge_index[1], pad_e])
  src2 = src1.reshape(NROWS, CW)
  dst2 = dst1.reshape(NROWS, CW)
  xp = jnp.pad(x, ((0, 0), (0, DP - x.shape[1])))
  W1p = jnp.pad(W1, ((0, DP - W1.shape[0]), (0, 0)))

  degp = _deg_kernel()(dst1)
  dinv_col, xs, dinvc_col, fiv_col, invc_row, mask_row = _stats_tc(
      degp[:, :NN].reshape(NC, NN, 1), batch.reshape(NN, 1), xp)

  p1 = _prop_kernel(DP)(_pad_rows(xs, NNP), src2, dst2)
  hs1 = _layer_tc(p1[:, :NN, :], xs, dinv_col, W1p, b1.reshape(1, HH),
                  relu=True, scale_out=True)

  hs1p = _pad_rows(hs1, NNP)
  p2lo = _prop_kernel(HH // 2)(hs1p[:, :HH // 2], src2, dst2)
  p2hi = _prop_kernel(HH // 2)(hs1p[:, HH // 2:], src2, dst2)
  h2 = _layer2_tc(p2lo[:, :NN, :], p2hi[:, :NN, :], hs1, dinv_col, W2,
                  b2.reshape(1, HH))

  Mp = _mbuild_kernel()(src1, dst1, _pad_rows(dinv_col.reshape(NN), NNP),
                        _pad_rows(dinvc_col.reshape(NN), NNP),
                        _pad_rows(fiv_col.reshape(NN), NNP))

  return _head_tc(Mp.reshape(NC, GG, NNP)[:, :, :NN], h2, batch.reshape(1, NN),
                  dinv_col.reshape(1, NN), invc_row.reshape(GG, 1),
                  mask_row.reshape(GG, 1), W3, b3.reshape(1, HH), Wl,
                  bl.reshape(1, 2))


# speedup vs baseline: 1.1914x; 1.1914x over previous
import functools

import jax
import jax.numpy as jnp
from jax import lax
from jax.experimental import pallas as pl
from jax.experimental.pallas import tpu as pltpu
from jax.experimental.pallas import tpu_sc as plsc

NN = 10000
NNP = 10240
EE = 320000
HH = 128
GG = 64
DP = 16
CW = 128
NROWS = 2560
EEP = NROWS * CW
NC = 2
NS = 16
NW = NC * NS
RW = NROWS // NW
RWC = (136, 24)
RWMAX = max(RWC)
CBASE = (0, NS * RWC[0])
ZR = 128
NPW = NNP // NS
MF = GG * NNP
MPW = MF // NS
MZC = 4096


def _sc_mesh():
  return plsc.VectorSubcoreMesh(core_axis_name="c", subcore_axis_name="s",
                                num_cores=NC, num_subcores=NS)


@functools.cache
def _deg_kernel():
  @functools.partial(
      pl.kernel,
      out_type=jax.ShapeDtypeStruct((NC, NNP), jnp.float32),
      mesh=_sc_mesh(),
      compiler_params=pltpu.CompilerParams(use_tc_tiling_on_sc=False),
      scratch_types=[
          pltpu.VMEM((RW * CW,), jnp.int32),
          pltpu.VMEM((RW * CW,), jnp.float32),
          pltpu.VMEM((NPW,), jnp.float32),
          pltpu.VMEM_SHARED((NNP,), jnp.float32),
      ],
  )
  def deg(dst_hbm, out_hbm, dstv, onesv, zb, acc):
    c = lax.axis_index("c")
    s = lax.axis_index("s")
    wid = c * NS + s

    def fill1(i, _):
      onesv[pl.ds(i * 16, 16)] = jnp.ones((16,), jnp.float32)
      return 0
    lax.fori_loop(0, RW * CW // 16, fill1, 0)

    def fill0(i, _):
      zb[pl.ds(i * 16, 16)] = jnp.zeros((16,), jnp.float32)
      return 0
    lax.fori_loop(0, NPW // 16, fill0, 0)

    pltpu.sync_copy(dst_hbm.at[pl.ds(wid * RW * CW, RW * CW)], dstv)
    pltpu.sync_copy(zb, acc.at[pl.ds(s * NPW, NPW)])
    plsc.subcore_barrier()

    pltpu.sync_copy(onesv, acc.at[dstv], add=True)

    plsc.subcore_barrier()
    pltpu.sync_copy(acc.at[pl.ds(s * NPW, NPW)],
                    out_hbm.at[c, pl.ds(s * NPW, NPW)])

  return deg


@functools.cache
def _prop_kernel(F):
  @functools.partial(
      pl.kernel,
      out_type=jax.ShapeDtypeStruct((NC, NNP, F), jnp.float32),
      mesh=_sc_mesh(),
      compiler_params=pltpu.CompilerParams(use_tc_tiling_on_sc=False),
      scratch_types=[
          pltpu.VMEM((RWMAX, CW), jnp.int32),
          pltpu.VMEM((RWMAX, CW), jnp.int32),
          pltpu.VMEM((CW, F), jnp.float32),
          pltpu.VMEM((CW, F), jnp.float32),
          pltpu.VMEM((CW, F), jnp.float32),
          pltpu.VMEM((CW, F), jnp.float32),
          pltpu.VMEM((ZR, F), jnp.float32),
          pltpu.VMEM_SHARED((NNP, F), jnp.float32),
          pltpu.SemaphoreType.DMA,
          pltpu.SemaphoreType.DMA,
          pltpu.SemaphoreType.DMA,
          pltpu.SemaphoreType.DMA,
          pltpu.SemaphoreType.DMA,
          pltpu.SemaphoreType.DMA,
          pltpu.SemaphoreType.DMA,
          pltpu.SemaphoreType.DMA,
      ],
  )
  def prop(h_hbm, src_hbm, dst_hbm, out_hbm, srcv, dstv, rb0, rb1, rb2, rb3,
           zb, acc, sg0, sg1, sg2, sg3, ss0, ss1, ss2, ss3):
    c = lax.axis_index("c")
    s = lax.axis_index("s")
    rbufs = (rb0, rb1, rb2, rb3)
    sgs = (sg0, sg1, sg2, sg3)
    sss = (ss0, ss1, ss2, ss3)

    def z0(i, _):
      def z1(k, _):
        zb[i, pl.ds(k * 16, 16)] = jnp.zeros((16,), jnp.float32)
        return 0
      lax.fori_loop(0, F // 16, z1, 0)
      return 0
    lax.fori_loop(0, ZR, z0, 0)

    def zc(t, _):
      pltpu.sync_copy(zb, acc.at[pl.ds(s * NPW + t * ZR, ZR)])
      return 0
    lax.fori_loop(0, NPW // ZR, zc, 0)

    base = jnp.where(c == 0, s * RWC[0], CBASE[1] + s * RWC[1])
    pltpu.sync_copy(src_hbm.at[pl.ds(base, RWMAX)], srcv)
    pltpu.sync_copy(dst_hbm.at[pl.ds(base, RWMAX)], dstv)
    plsc.subcore_barrier()

    def run(cc):
      rows = RWC[cc]

      pltpu.async_copy(h_hbm.at[srcv.at[0]], rb0, sg0)
      pltpu.async_copy(h_hbm.at[srcv.at[1]], rb1, sg1)

      def body(j, _):
        def make_branch(b):
          b2 = (b + 2) % 4

          def br():
            pltpu.make_async_copy(h_hbm.at[srcv.at[j]], rbufs[b],
                                  sgs[b]).wait()
            pltpu.async_copy(rbufs[b], acc.at[dstv.at[j]], sss[b], add=True)

            @pl.when(j + 2 < rows)
            def _():
              @pl.when(j >= 2)
              def _():
                pltpu.make_async_copy(rbufs[b2], acc.at[dstv.at[j - 2]],
                                      sss[b2]).wait()
              pltpu.async_copy(h_hbm.at[srcv.at[j + 2]], rbufs[b2], sgs[b2])
          return br

        for b in range(4):
          pl.when(j % 4 == b)(make_branch(b))
        return 0
      lax.fori_loop(0, rows, body, 0)

      for b in range(4):
        pltpu.make_async_copy(rbufs[(rows - 4 + b) % 4],
                              acc.at[dstv.at[rows - 4 + b]],
                              sss[(rows - 4 + b) % 4]).wait()

    for cc in range(NC):
      pl.when(c == cc)(lambda cc=cc: run(cc))

    plsc.subcore_barrier()

    def wo(t, _):
      off = s * NPW + t * ZR
      pltpu.sync_copy(acc.at[pl.ds(off, ZR)], out_hbm.at[c, pl.ds(off, ZR)])
      return 0
    lax.fori_loop(0, NPW // ZR, wo, 0)

  return prop


@functools.cache
def _mbuild_kernel():
  @functools.partial(
      pl.kernel,
      out_type=jax.ShapeDtypeStruct((NC, MF), jnp.float32),
      mesh=_sc_mesh(),
      compiler_params=pltpu.CompilerParams(use_tc_tiling_on_sc=False,
                                           needs_layout_passes=False),
      scratch_types=[
          pltpu.VMEM((NNP,), jnp.float32),
          pltpu.VMEM((NNP,), jnp.float32),
          pltpu.VMEM((NNP,), jnp.int32),
          pltpu.VMEM((RW * CW,), jnp.int32),
          pltpu.VMEM((RW * CW,), jnp.int32),
          pltpu.VMEM((RW * CW,), jnp.float32),
          pltpu.VMEM((RW * CW,), jnp.int32),
          pltpu.VMEM((MZC,), jnp.float32),
          pltpu.VMEM_SHARED((MF,), jnp.float32),
      ],
  )
  def mbuild(src_hbm, dst_hbm, dinv_hbm, dinvc_hbm, fiv_hbm, out_hbm,
             dinvv, dinvcv, fivv, srcv, dstv, vals, fidx, zb, macc):
    c = lax.axis_index("c")
    s = lax.axis_index("s")

    wid = c * NS + s

    def fz(i, _):
      zb[pl.ds(i * 16, 16)] = jnp.zeros((16,), jnp.float32)
      return 0
    lax.fori_loop(0, MZC // 16, fz, 0)

    pltpu.sync_copy(dinv_hbm, dinvv)
    pltpu.sync_copy(dinvc_hbm, dinvcv)
    pltpu.sync_copy(fiv_hbm, fivv)
    pltpu.sync_copy(src_hbm.at[pl.ds(wid * RW * CW, RW * CW)], srcv)
    pltpu.sync_copy(dst_hbm.at[pl.ds(wid * RW * CW, RW * CW)], dstv)

    def zc(t, _):
      pltpu.sync_copy(zb, macc.at[pl.ds(s * MPW + t * MZC, MZC)])
      return 0
    lax.fori_loop(0, MPW // MZC, zc, 0)

    plsc.subcore_barrier()

    def body(i, _):
      sl = pl.ds(i * 16, 16)
      s16 = srcv[sl]
      d16 = dstv[sl]
      a = plsc.load_gather(dinvv, [s16])
      b = plsc.load_gather(dinvcv, [d16])
      f = plsc.load_gather(fivv, [d16])
      vals[sl] = a * b
      fidx[sl] = f + s16
      return 0
    lax.fori_loop(0, RW * CW // 16, body, 0)
    pltpu.sync_copy(vals, macc.at[fidx], add=True)

    plsc.subcore_barrier()

    def wo(t, _):
      off = s * MPW + t * MZC
      pltpu.sync_copy(macc.at[pl.ds(off, MZC)], out_hbm.at[c, pl.ds(off, MZC)])
      return 0
    lax.fori_loop(0, MPW // MZC, wo, 0)

  return mbuild


def _stats_tc(degp, batch_col, xp):
  def body(degp_ref, b_ref, x_ref, dinv_ref, xs_ref, dinvc_ref, fiv_ref,
           invc_ref, mask_ref):
    deg = degp_ref[0] + degp_ref[1] + 1.0
    dinv = lax.rsqrt(deg)
    b = b_ref[:]
    io = lax.broadcasted_iota(jnp.int32, (NN, GG), 1)
    oh = (b == io).astype(jnp.float32)
    cnt = jnp.sum(oh, axis=0, keepdims=True)
    invc = 1.0 / jnp.maximum(cnt, 1.0)
    ipn = lax.dot_general(oh, invc, (((1,), (1,)), ((), ())),
                          preferred_element_type=jnp.float32)
    dinv_ref[:] = dinv
    xs_ref[:] = x_ref[:] * dinv
    dinvc_ref[:] = dinv * ipn
    fiv_ref[:] = b * NNP
    invc_ref[:] = invc
    mask_ref[:] = (cnt > 0.0).astype(jnp.float32)

  return pl.pallas_call(
      body,
      out_shape=(
          jax.ShapeDtypeStruct((NN, 1), jnp.float32),
          jax.ShapeDtypeStruct((NN, DP), jnp.float32),
          jax.ShapeDtypeStruct((NN, 1), jnp.float32),
          jax.ShapeDtypeStruct((NN, 1), jnp.int32),
          jax.ShapeDtypeStruct((1, GG), jnp.float32),
          jax.ShapeDtypeStruct((1, GG), jnp.float32),
      ),
  )(degp, batch_col, xp)


def _layer_tc(p, hprev, dinv_col, Wm, bias, relu, scale_out):
  Fout = HH if Wm is None else Wm.shape[1]

  def body(p_ref, h_ref, d_ref, *rest):
    if Wm is None:
      b_ref, o_ref = rest
    else:
      w_ref, b_ref, o_ref = rest
    z = (p_ref[0] + p_ref[1] + h_ref[:]) * d_ref[:]
    if Wm is not None:
      z = jnp.dot(z, w_ref[:], preferred_element_type=jnp.float32)
    h = z + b_ref[:]
    if relu:
      h = jnp.maximum(h, 0.0)
    if scale_out:
      h = h * d_ref[:]
    o_ref[:] = h

  args = (p, hprev, dinv_col) + (() if Wm is None else (Wm,)) + (bias,)
  return pl.pallas_call(
      body, out_shape=jax.ShapeDtypeStruct((NN, Fout), jnp.float32)
  )(*args)


def _layer2_tc(p_lo, p_hi, hprev, dinv_col, Wm, bias):
  def body(plo_ref, phi_ref, h_ref, d_ref, w_ref, b_ref, o_ref):
    p = jnp.concatenate([plo_ref[0] + plo_ref[1], phi_ref[0] + phi_ref[1]],
                        axis=1)
    z = (p + h_ref[:]) * d_ref[:]
    h = jnp.dot(z, w_ref[:], preferred_element_type=jnp.float32) + b_ref[:]
    o_ref[:] = jnp.maximum(h, 0.0)

  return pl.pallas_call(
      body, out_shape=jax.ShapeDtypeStruct((NN, HH), jnp.float32)
  )(p_lo, p_hi, hprev, dinv_col, Wm, bias)


def _head_tc(Mp, h2, batch_row, dinv_row, invc_col, mask_col, W3, b3r, Wl, blr):
  def body(mp_ref, h2_ref, b_ref, dv_ref, ic_ref, mk_ref, w3_ref, b3_ref,
           wl_ref, bl_ref, o_ref):
    io = lax.broadcasted_iota(jnp.int32, (GG, NN), 0)
    oh = (b_ref[:] == io).astype(jnp.float32)
    dv = dv_ref[:]
    mtot = mp_ref[0] + mp_ref[1] + oh * (dv * dv) * ic_ref[:]
    pooled = jnp.dot(mtot, h2_ref[:], preferred_element_type=jnp.float32)
    pooled = (jnp.dot(pooled, w3_ref[:], preferred_element_type=jnp.float32)
              + mk_ref[:] * b3_ref[:])
    o_ref[:] = (jnp.dot(pooled, wl_ref[:], preferred_element_type=jnp.float32)
                + bl_ref[:])

  return pl.pallas_call(
      body, out_shape=jax.ShapeDtypeStruct((GG, 2), jnp.float32)
  )(Mp, h2, batch_row, dinv_row, invc_col, mask_col, W3, b3r, Wl, blr)


def _pad_rows(a, rows):
  return jnp.pad(a, ((0, rows - a.shape[0]),) + ((0, 0),) * (a.ndim - 1))


def kernel(x, edge_index, batch, W1, b1, W2, b2, W3, b3, Wl, bl):
  pad_e = jnp.full((EEP - EE,), NN, jnp.int32)
  src1 = jnp.concatenate([edge_index[0], pad_e])
  dst1 = jnp.concatenate([edge_index[1], pad_e])
  src2 = src1.reshape(NROWS, CW)
  dst2 = dst1.reshape(NROWS, CW)
  xp = jnp.pad(x, ((0, 0), (0, DP - x.shape[1])))
  W1p = jnp.pad(W1, ((0, DP - W1.shape[0]), (0, 0)))

  degp = _deg_kernel()(dst1)
  dinv_col, xs, dinvc_col, fiv_col, invc_row, mask_row = _stats_tc(
      degp[:, :NN].reshape(NC, NN, 1), batch.reshape(NN, 1), xp)

  p1 = _prop_kernel(DP)(_pad_rows(xs, NNP), src2, dst2)
  hs1 = _layer_tc(p1[:, :NN, :], xs, dinv_col, W1p, b1.reshape(1, HH),
                  relu=True, scale_out=True)

  hs1p = _pad_rows(hs1, NNP)
  p2lo = _prop_kernel(HH // 2)(hs1p[:, :HH // 2], src2, dst2)
  p2hi = _prop_kernel(HH // 2)(hs1p[:, HH // 2:], src2, dst2)
  h2 = _layer2_tc(p2lo[:, :NN, :], p2hi[:, :NN, :], hs1, dinv_col, W2,
                  b2.reshape(1, HH))

  Mp = _mbuild_kernel()(src1, dst1, _pad_rows(dinv_col.reshape(NN), NNP),
                        _pad_rows(dinvc_col.reshape(NN), NNP),
                        _pad_rows(fiv_col.reshape(NN), NNP))

  return _head_tc(Mp.reshape(NC, GG, NNP)[:, :, :NN], h2, batch.reshape(1, NN),
                  dinv_col.reshape(1, NN), invc_row.reshape(GG, 1),
                  mask_row.reshape(GG, 1), W3, b3.reshape(1, HH), Wl,
                  bl.reshape(1, 2))
